# Initial kernel scaffold; baseline (speedup 1.0000x reference)
import jax
import jax.numpy as jnp
from jax.experimental import pallas as pl


def kernel(x, W, b):
    raise NotImplementedError("write your pallas kernel here")


# speedup vs baseline: 2.9074x; 2.9074x over previous
import jax
import jax.numpy as jnp
from jax import lax
from jax.experimental import pallas as pl

DIM = 4096
NUM_EXPERTS = 64
TOP_K = 8
TOKENS = 16384
CAPACITY_FACTOR = 1.0

BLOCK_T = 256


def _router_kernel(x_ref, wt_ref, b_ref, logits_ref, idx_ref, wts_ref, mask_ref):
    x = x_ref[...]
    wt = wt_ref[...]
    b = b_ref[...]
    logits = lax.dot_general(
        x, wt, (((1,), (0,)), ((), ())), preferred_element_type=jnp.float32
    ) + b
    logits_ref[...] = logits

    iota = lax.broadcasted_iota(jnp.int32, logits.shape, 1)
    work = logits
    mask = jnp.zeros_like(logits)
    vals = []
    idxs = []
    for _ in range(TOP_K):
        m = jnp.max(work, axis=1, keepdims=True)
        is_max = work == m
        sel_idx = jnp.min(jnp.where(is_max, iota, NUM_EXPERTS),
                          axis=1, keepdims=True)
        sel = iota == sel_idx
        vals.append(m)
        idxs.append(sel_idx)
        mask = jnp.where(sel, 1.0, mask)
        work = jnp.where(sel, -jnp.inf, work)
    mask_ref[...] = mask

    tv = jnp.concatenate(vals, axis=1)
    ti = jnp.concatenate(idxs, axis=1)
    e = jnp.exp(tv - tv[:, 0:1])
    wts_ref[...] = e / jnp.sum(e, axis=1, keepdims=True)
    idx_ref[...] = ti


def kernel(x, W, b):
    wt = W.T
    b2 = b.reshape(1, NUM_EXPERTS)
    grid = (TOKENS // BLOCK_T,)
    logits, idx, wts, mask = pl.pallas_call(
        _router_kernel,
        grid=grid,
        in_specs=[
            pl.BlockSpec((BLOCK_T, DIM), lambda i: (i, 0)),
            pl.BlockSpec((DIM, NUM_EXPERTS), lambda i: (0, 0)),
            pl.BlockSpec((1, NUM_EXPERTS), lambda i: (0, 0)),
        ],
        out_specs=[
            pl.BlockSpec((BLOCK_T, NUM_EXPERTS), lambda i: (i, 0)),
            pl.BlockSpec((BLOCK_T, TOP_K), lambda i: (i, 0)),
            pl.BlockSpec((BLOCK_T, TOP_K), lambda i: (i, 0)),
            pl.BlockSpec((BLOCK_T, NUM_EXPERTS), lambda i: (i, 0)),
        ],
        out_shape=[
            jax.ShapeDtypeStruct((TOKENS, NUM_EXPERTS), jnp.float32),
            jax.ShapeDtypeStruct((TOKENS, TOP_K), jnp.int32),
            jax.ShapeDtypeStruct((TOKENS, TOP_K), jnp.float32),
            jax.ShapeDtypeStruct((TOKENS, NUM_EXPERTS), jnp.float32),
        ],
    )(x, wt, b2)
    capacity = min(TOKENS, int(CAPACITY_FACTOR * TOKENS / NUM_EXPERTS * TOP_K))
    return (logits, idx, wts, mask, jnp.int32(capacity))


# speedup vs baseline: 3.3904x; 1.1661x over previous
import jax
import jax.numpy as jnp
from jax import lax
from jax.experimental import pallas as pl

DIM = 4096
NUM_EXPERTS = 64
TOP_K = 8
TOKENS = 16384
CAPACITY_FACTOR = 1.0

BLOCK_T = 256


def _router_kernel(x_ref, wt_ref, b_ref, logits_ref, idx_ref, wts_ref, mask_ref):
    x = x_ref[...]
    wt = wt_ref[...]
    b = b_ref[...]
    logits = lax.dot_general(
        x, wt, (((1,), (0,)), ((), ())), preferred_element_type=jnp.float32
    ) + b
    logits_ref[...] = logits

    iota_f = lax.broadcasted_iota(jnp.int32, logits.shape, 1).astype(jnp.float32)
    work = logits
    vals = []
    idxs = []
    for _ in range(TOP_K):
        m = jnp.max(work, axis=1, keepdims=True)
        cand = jnp.where(work == m, iota_f, float(NUM_EXPERTS))
        idx_f = jnp.min(cand, axis=1, keepdims=True)
        work = jnp.where(iota_f == idx_f, -jnp.inf, work)
        vals.append(m)
        idxs.append(idx_f)
    mask_ref[...] = (work == -jnp.inf).astype(jnp.float32)

    tv = jnp.concatenate(vals, axis=1)
    ti = jnp.concatenate(idxs, axis=1)
    e = jnp.exp(tv - tv[:, 0:1])
    wts_ref[...] = e / jnp.sum(e, axis=1, keepdims=True)
    idx_ref[...] = ti.astype(jnp.int32)


def kernel(x, W, b):
    wt = W.T
    b2 = b.reshape(1, NUM_EXPERTS)
    grid = (TOKENS // BLOCK_T,)
    logits, idx, wts, mask = pl.pallas_call(
        _router_kernel,
        grid=grid,
        in_specs=[
            pl.BlockSpec((BLOCK_T, DIM), lambda i: (i, 0)),
            pl.BlockSpec((DIM, NUM_EXPERTS), lambda i: (0, 0)),
            pl.BlockSpec((1, NUM_EXPERTS), lambda i: (0, 0)),
        ],
        out_specs=[
            pl.BlockSpec((BLOCK_T, NUM_EXPERTS), lambda i: (i, 0)),
            pl.BlockSpec((BLOCK_T, TOP_K), lambda i: (i, 0)),
            pl.BlockSpec((BLOCK_T, TOP_K), lambda i: (i, 0)),
            pl.BlockSpec((BLOCK_T, NUM_EXPERTS), lambda i: (i, 0)),
        ],
        out_shape=[
            jax.ShapeDtypeStruct((TOKENS, NUM_EXPERTS), jnp.float32),
            jax.ShapeDtypeStruct((TOKENS, TOP_K), jnp.int32),
            jax.ShapeDtypeStruct((TOKENS, TOP_K), jnp.float32),
            jax.ShapeDtypeStruct((TOKENS, NUM_EXPERTS), jnp.float32),
        ],
    )(x, wt, b2)
    capacity = min(TOKENS, int(CAPACITY_FACTOR * TOKENS / NUM_EXPERTS * TOP_K))
    return (logits, idx, wts, mask, jnp.int32(capacity))


# speedup vs baseline: 4.5770x; 1.3500x over previous
import jax
import jax.numpy as jnp
from jax import lax
from jax.experimental import pallas as pl

DIM = 4096
NUM_EXPERTS = 64
TOP_K = 8
TOKENS = 16384
CAPACITY_FACTOR = 1.0

BLOCK_T = 1024


def _router_kernel(x_ref, wt_ref, b_ref, logits_ref, idx_ref, wts_ref, mask_ref):
    x = x_ref[...]
    wt = wt_ref[...]
    b = b_ref[...]
    logits = lax.dot_general(
        x, wt, (((1,), (0,)), ((), ())), preferred_element_type=jnp.float32
    ) + b
    logits_ref[...] = logits

    iota_f = lax.broadcasted_iota(jnp.int32, logits.shape, 1).astype(jnp.float32)
    work = logits
    vals = []
    idxs = []
    for _ in range(TOP_K):
        m = jnp.max(work, axis=1, keepdims=True)
        cand = jnp.where(work == m, iota_f, float(NUM_EXPERTS))
        idx_f = jnp.min(cand, axis=1, keepdims=True)
        work = jnp.where(iota_f == idx_f, -jnp.inf, work)
        vals.append(m)
        idxs.append(idx_f)
    mask_ref[...] = (work == -jnp.inf).astype(jnp.float32)

    tv = jnp.concatenate(vals, axis=1)
    ti = jnp.concatenate(idxs, axis=1)
    e = jnp.exp(tv - tv[:, 0:1])
    wts_ref[...] = e / jnp.sum(e, axis=1, keepdims=True)
    idx_ref[...] = ti.astype(jnp.int32)


def kernel(x, W, b):
    wt = W.T
    b2 = b.reshape(1, NUM_EXPERTS)
    grid = (TOKENS // BLOCK_T,)
    logits, idx, wts, mask = pl.pallas_call(
        _router_kernel,
        grid=grid,
        in_specs=[
            pl.BlockSpec((BLOCK_T, DIM), lambda i: (i, 0)),
            pl.BlockSpec((DIM, NUM_EXPERTS), lambda i: (0, 0)),
            pl.BlockSpec((1, NUM_EXPERTS), lambda i: (0, 0)),
        ],
        out_specs=[
            pl.BlockSpec((BLOCK_T, NUM_EXPERTS), lambda i: (i, 0)),
            pl.BlockSpec((BLOCK_T, TOP_K), lambda i: (i, 0)),
            pl.BlockSpec((BLOCK_T, TOP_K), lambda i: (i, 0)),
            pl.BlockSpec((BLOCK_T, NUM_EXPERTS), lambda i: (i, 0)),
        ],
        out_shape=[
            jax.ShapeDtypeStruct((TOKENS, NUM_EXPERTS), jnp.float32),
            jax.ShapeDtypeStruct((TOKENS, TOP_K), jnp.int32),
            jax.ShapeDtypeStruct((TOKENS, TOP_K), jnp.float32),
            jax.ShapeDtypeStruct((TOKENS, NUM_EXPERTS), jnp.float32),
        ],
    )(x, wt, b2)
    capacity = min(TOKENS, int(CAPACITY_FACTOR * TOKENS / NUM_EXPERTS * TOP_K))
    return (logits, idx, wts, mask, jnp.int32(capacity))
